# baseline (speedup 1.0000x reference)
import jax
import jax.numpy as jnp
from jax import lax
from jax.experimental import pallas as pl
from jax.experimental.pallas import tpu as pltpu

NUM_GRAPHS = 64
NUM_VIRTUAL = 4
N_NODES = 10000
HIDDEN = 128

BLOCK_ROWS = 1000
NUM_BLOCKS = N_NODES // BLOCK_ROWS


def _fused_kernel(x_ref, batch_ref, W_enc_ref, b_enc_ref, W1_ref, b1_ref,
                  W2_ref, b2_ref, Wp1_ref, bp1_ref, Wp2_ref, bp2_ref,
                  out_ref, acc_ref):
    i = pl.program_id(0)

    @pl.when(i == 0)
    def _init():
        acc_ref[...] = jnp.zeros_like(acc_ref)

    xb = x_ref[...]
    nf = jnp.maximum(jnp.dot(xb, W_enc_ref[...]) + b_enc_ref[...], 0.0)

    bb = batch_ref[0, 0, :]
    onehot_t = (lax.broadcasted_iota(jnp.int32, (NUM_GRAPHS, BLOCK_ROWS), 0)
                == bb[None, :]).astype(jnp.float32)
    acc_ref[...] += jnp.dot(onehot_t, nf)

    @pl.when(i == NUM_BLOCKS - 1)
    def _finish():
        seg = acc_ref[...] * (1.0 / NUM_VIRTUAL)
        h = jnp.maximum(jnp.dot(seg, W1_ref[...]) + b1_ref[...], 0.0)
        gf = jnp.dot(h, W2_ref[...]) + b2_ref[...]
        p = jnp.maximum(jnp.dot(gf, Wp1_ref[...]) + bp1_ref[...], 0.0)
        out_ref[...] = jnp.dot(p, Wp2_ref[...]) + bp2_ref[...]


def kernel(x, edge_index, batch, W_enc, b_enc, W1, b1, W2, b2, Wp1, bp1,
           Wp2, bp2):
    del edge_index
    batch3 = batch.reshape(NUM_BLOCKS, 1, BLOCK_ROWS)

    full = lambda shape: pl.BlockSpec(shape, lambda i: (0,) * len(shape))
    out = pl.pallas_call(
        _fused_kernel,
        grid=(NUM_BLOCKS,),
        in_specs=[
            pl.BlockSpec((BLOCK_ROWS, HIDDEN), lambda i: (i, 0)),
            pl.BlockSpec((1, 1, BLOCK_ROWS), lambda i: (i, 0, 0)),
            full((HIDDEN, HIDDEN)),
            full((1, HIDDEN)),
            full((HIDDEN, HIDDEN)),
            full((1, HIDDEN)),
            full((HIDDEN, HIDDEN)),
            full((1, HIDDEN)),
            full((HIDDEN, HIDDEN)),
            full((1, HIDDEN)),
            full((HIDDEN, 1)),
            full((1, 1)),
        ],
        out_specs=pl.BlockSpec((NUM_GRAPHS, 1), lambda i: (0, 0)),
        out_shape=jax.ShapeDtypeStruct((NUM_GRAPHS, 1), jnp.float32),
        scratch_shapes=[pltpu.VMEM((NUM_GRAPHS, HIDDEN), jnp.float32)],
    )(x, batch3, W_enc, b_enc.reshape(1, HIDDEN), W1, b1.reshape(1, HIDDEN),
      W2, b2.reshape(1, HIDDEN), Wp1, bp1.reshape(1, HIDDEN), Wp2,
      bp2.reshape(1, 1))
    return out


# speedup vs baseline: 1.4519x; 1.4519x over previous
import jax
import jax.numpy as jnp
from jax import lax
from jax.experimental import pallas as pl
from jax.experimental.pallas import tpu as pltpu

NUM_GRAPHS = 64
NUM_VIRTUAL = 4
N_NODES = 10000
HIDDEN = 128

BLOCK_ROWS = 10000
NUM_BLOCKS = N_NODES // BLOCK_ROWS


def _fused_kernel(x_ref, batch_ref, W_enc_ref, b_enc_ref, W1_ref, b1_ref,
                  W2_ref, b2_ref, Wp1_ref, bp1_ref, Wp2_ref, bp2_ref,
                  out_ref, acc_ref):
    i = pl.program_id(0)

    @pl.when(i == 0)
    def _init():
        acc_ref[...] = jnp.zeros_like(acc_ref)

    xb = x_ref[...]
    nf = jnp.maximum(jnp.dot(xb, W_enc_ref[...]) + b_enc_ref[...], 0.0)

    bb = batch_ref[0, 0, :]
    onehot_t = (lax.broadcasted_iota(jnp.int32, (NUM_GRAPHS, BLOCK_ROWS), 0)
                == bb[None, :]).astype(jnp.float32)
    acc_ref[...] += jnp.dot(onehot_t, nf)

    @pl.when(i == NUM_BLOCKS - 1)
    def _finish():
        seg = acc_ref[...] * (1.0 / NUM_VIRTUAL)
        h = jnp.maximum(jnp.dot(seg, W1_ref[...]) + b1_ref[...], 0.0)
        gf = jnp.dot(h, W2_ref[...]) + b2_ref[...]
        p = jnp.maximum(jnp.dot(gf, Wp1_ref[...]) + bp1_ref[...], 0.0)
        out_ref[...] = jnp.dot(p, Wp2_ref[...]) + bp2_ref[...]


def kernel(x, edge_index, batch, W_enc, b_enc, W1, b1, W2, b2, Wp1, bp1,
           Wp2, bp2):
    del edge_index
    batch3 = batch.reshape(NUM_BLOCKS, 1, BLOCK_ROWS)

    full = lambda shape: pl.BlockSpec(shape, lambda i: (0,) * len(shape))
    out = pl.pallas_call(
        _fused_kernel,
        grid=(NUM_BLOCKS,),
        in_specs=[
            pl.BlockSpec((BLOCK_ROWS, HIDDEN), lambda i: (i, 0)),
            pl.BlockSpec((1, 1, BLOCK_ROWS), lambda i: (i, 0, 0)),
            full((HIDDEN, HIDDEN)),
            full((1, HIDDEN)),
            full((HIDDEN, HIDDEN)),
            full((1, HIDDEN)),
            full((HIDDEN, HIDDEN)),
            full((1, HIDDEN)),
            full((HIDDEN, HIDDEN)),
            full((1, HIDDEN)),
            full((HIDDEN, 1)),
            full((1, 1)),
        ],
        out_specs=pl.BlockSpec((NUM_GRAPHS, 1), lambda i: (0, 0)),
        out_shape=jax.ShapeDtypeStruct((NUM_GRAPHS, 1), jnp.float32),
        scratch_shapes=[pltpu.VMEM((NUM_GRAPHS, HIDDEN), jnp.float32)],
    )(x, batch3, W_enc, b_enc.reshape(1, HIDDEN), W1, b1.reshape(1, HIDDEN),
      W2, b2.reshape(1, HIDDEN), Wp1, bp1.reshape(1, HIDDEN), Wp2,
      bp2.reshape(1, 1))
    return out
